# Initial kernel scaffold; baseline (speedup 1.0000x reference)
import jax
import jax.numpy as jnp
from jax.experimental import pallas as pl


def kernel(x, edge_index, W0_z, b0_z, W0_r, b0_r, W0_h, b0_h, W1_z, b1_z, W1_r, b1_r, W1_h, b1_h):
    raise NotImplementedError("write your pallas kernel here")


# speedup vs baseline: 13.7422x; 13.7422x over previous
import functools

import jax
import jax.numpy as jnp
from jax import lax
from jax.experimental import pallas as pl
from jax.experimental.pallas import tpu as pltpu
from jax.experimental.pallas import tpu_sc as plsc

N = 10000
NP = 10240
E = 320000
D = 128
DH = D // 2

NUM_TILES = 16
EDGES_PER_TILE = E // NUM_TILES
CHUNK = 80
NCHUNK = EDGES_PER_TILE // CHUNK
ROWS_PER_TILE = NP // NUM_TILES

RBLK = 1280

_mesh = plsc.VectorSubcoreMesh(core_axis_name="c", subcore_axis_name="s")
_sc_params = pltpu.CompilerParams(use_tc_tiling_on_sc=False)


@functools.partial(
    pl.kernel,
    out_type=jax.ShapeDtypeStruct((2, NP), jnp.float32),
    mesh=_mesh,
    scratch_types=[
        pltpu.VMEM((NCHUNK, CHUNK), jnp.int32),
        pltpu.VMEM((CHUNK,), jnp.float32),
        pltpu.VMEM_SHARED((NP,), jnp.float32),
        pltpu.SemaphoreType.DMA,
    ],
    compiler_params=_sc_params,
)
def _deg_kernel(ei3, zeros_rows, ones_hbm, deg2, idx, ones_v, dacc, sem):
    c = lax.axis_index("c")
    s = lax.axis_index("s")

    def run(r):
        pltpu.sync_copy(ones_hbm, ones_v)
        pltpu.sync_copy(zeros_rows, dacc.at[pl.ds(s * ROWS_PER_TILE, ROWS_PER_TILE)])
        plsc.subcore_barrier()
        pltpu.sync_copy(ei3.at[r, s], idx)

        def chunk(i, carry):
            pltpu.sync_copy(ones_v, dacc.at[idx.at[i]], add=True)
            return carry

        lax.fori_loop(0, NCHUNK, chunk, 0)
        plsc.subcore_barrier()
        pltpu.sync_copy(dacc.at[pl.ds(s * ROWS_PER_TILE, ROWS_PER_TILE)],
                        deg2.at[r, pl.ds(s * ROWS_PER_TILE, ROWS_PER_TILE)])

    @pl.when(c == 0)
    def _():
        run(0)

    @pl.when(c == 1)
    def _():
        run(1)


_half = jax.ShapeDtypeStruct((NP, DH), jnp.float32)


@functools.partial(
    pl.kernel,
    out_type=(_half, _half, _half, _half),
    mesh=_mesh,
    scratch_types=[
        pltpu.VMEM((NCHUNK, CHUNK), jnp.int32),
        pltpu.VMEM((NCHUNK, CHUNK), jnp.int32),
        pltpu.VMEM((CHUNK, DH), jnp.float32),
        pltpu.VMEM_SHARED((NP, DH), jnp.float32),
        pltpu.SemaphoreType.DMA,
    ],
    compiler_params=_sc_params,
)
def _prop_kernel(xo_lo, xo_hi, xi_lo, xi_hi, ei3, zrows,
                 to_lo, to_hi, ti_lo, ti_hi,
                 idx_src, idx_dst, rows, accum, sem):
    c = lax.axis_index("c")
    s = lax.axis_index("s")

    def run(tables, src_row, dst_row, outs):
        pltpu.sync_copy(ei3.at[src_row, s], idx_src)
        pltpu.sync_copy(ei3.at[dst_row, s], idx_dst)
        for table, out in zip(tables, outs):
            for k in range(ROWS_PER_TILE // CHUNK):
                pltpu.sync_copy(
                    zrows, accum.at[pl.ds(s * ROWS_PER_TILE + k * CHUNK, CHUNK)])
            plsc.subcore_barrier()

            def chunk(i, carry):
                pltpu.async_copy(table.at[idx_src.at[i]], rows, sem).wait()
                pltpu.sync_copy(rows, accum.at[idx_dst.at[i]], add=True)
                return carry

            lax.fori_loop(0, NCHUNK, chunk, 0)
            plsc.subcore_barrier()
            pltpu.sync_copy(accum.at[pl.ds(s * ROWS_PER_TILE, ROWS_PER_TILE)],
                            out.at[pl.ds(s * ROWS_PER_TILE, ROWS_PER_TILE)])

    @pl.when(c == 0)
    def _():
        run((xo_lo, xo_hi), 0, 1, (to_lo, to_hi))

    @pl.when(c == 1)
    def _():
        run((xi_lo, xi_hi), 1, 0, (ti_lo, ti_hi))


def _inv(d):
    return jnp.where(d > 0, 1.0 / d, 0.0)


def _prescale_body(x_ref, do_ref, di_ref, xolo_ref, xohi_ref, xilo_ref, xihi_ref):
    xv = x_ref[...]
    xo = xv * _inv(do_ref[...])
    xi = xv * _inv(di_ref[...])
    xolo_ref[...] = xo[:, :DH]
    xohi_ref[...] = xo[:, DH:]
    xilo_ref[...] = xi[:, :DH]
    xihi_ref[...] = xi[:, DH:]


def _gates(x_ref, tol_ref, toh_ref, til_ref, tih_ref, w_ref, b_ref):
    a = jnp.concatenate([x_ref[...], tol_ref[...], toh_ref[...],
                         til_ref[...], tih_ref[...]], axis=1)
    u = jnp.dot(a, w_ref[...], preferred_element_type=jnp.float32) + b_ref[...]
    z = jax.nn.sigmoid(u[:, :D])
    h = jnp.tanh(u[:, D:])
    return (1.0 - z) * h


def _gate_mid_body(x_ref, tol_ref, toh_ref, til_ref, tih_ref, w_ref, b_ref,
                   do_ref, di_ref,
                   xn_ref, xolo_ref, xohi_ref, xilo_ref, xihi_ref):
    xn = _gates(x_ref, tol_ref, toh_ref, til_ref, tih_ref, w_ref, b_ref)
    xn_ref[...] = xn
    xo = xn * _inv(do_ref[...])
    xi = xn * _inv(di_ref[...])
    xolo_ref[...] = xo[:, :DH]
    xohi_ref[...] = xo[:, DH:]
    xilo_ref[...] = xi[:, :DH]
    xihi_ref[...] = xi[:, DH:]


def _gate_final_body(x_ref, tol_ref, toh_ref, til_ref, tih_ref, w_ref, b_ref,
                     xn_ref):
    xn_ref[...] = _gates(x_ref, tol_ref, toh_ref, til_ref, tih_ref, w_ref, b_ref)


_rows_spec = pl.BlockSpec((RBLK, D), lambda i: (i, 0))
_hrows_spec = pl.BlockSpec((RBLK, DH), lambda i: (i, 0))
_col_spec = pl.BlockSpec((RBLK, 1), lambda i: (i, 0))
_w_spec = pl.BlockSpec((3 * D, 2 * D), lambda i: (0, 0))
_b_spec = pl.BlockSpec((1, 2 * D), lambda i: (0, 0))
_f32 = jnp.float32
_half_out = jax.ShapeDtypeStruct((NP, DH), _f32)
_full_out = jax.ShapeDtypeStruct((NP, D), _f32)

_prescale_call = pl.pallas_call(
    _prescale_body,
    grid=(NP // RBLK,),
    in_specs=[_rows_spec, _col_spec, _col_spec],
    out_specs=(_hrows_spec,) * 4,
    out_shape=(_half_out,) * 4,
)

_gate_mid_call = pl.pallas_call(
    _gate_mid_body,
    grid=(NP // RBLK,),
    in_specs=[_rows_spec] + [_hrows_spec] * 4 + [_w_spec, _b_spec,
                                                 _col_spec, _col_spec],
    out_specs=(_rows_spec,) + (_hrows_spec,) * 4,
    out_shape=(_full_out,) + (_half_out,) * 4,
)

_gate_final_call = pl.pallas_call(
    _gate_final_body,
    grid=(NP // RBLK,),
    in_specs=[_rows_spec] + [_hrows_spec] * 4 + [_w_spec, _b_spec],
    out_specs=_rows_spec,
    out_shape=_full_out,
)


def _pack_weights(Wz, bz, Wh, bh):
    wz = jnp.concatenate([Wz[0, 0, :D] + Wz[1, 0, :D], Wz[0, 1, :D], Wz[1, 1, :D]], axis=0)
    wh = jnp.concatenate([Wh[0, 0, :D] + Wh[1, 0, :D], Wh[0, 1, :D], Wh[1, 1, :D]], axis=0)
    w = jnp.concatenate([wz, wh], axis=1)
    b = jnp.concatenate([bz, bh])[None, :]
    return w, b


def kernel(x, edge_index, W0_z, b0_z, W0_r, b0_r, W0_h, b0_h,
           W1_z, b1_z, W1_r, b1_r, W1_h, b1_h):
    ei3 = edge_index.astype(jnp.int32).reshape(2, NUM_TILES, NCHUNK, CHUNK)
    x_pad = jnp.zeros((NP, D), jnp.float32).at[:N].set(x)
    w0, bc0 = _pack_weights(W0_z, b0_z, W0_h, b0_h)
    w1, bc1 = _pack_weights(W1_z, b1_z, W1_h, b1_h)

    zeros_rows = jnp.zeros((ROWS_PER_TILE,), jnp.float32)
    ones_hbm = jnp.ones((CHUNK,), jnp.float32)
    zrows = jnp.zeros((CHUNK, DH), jnp.float32)

    deg2 = _deg_kernel(ei3, zeros_rows, ones_hbm)
    dego = deg2[0].reshape(NP, 1)
    degi = deg2[1].reshape(NP, 1)

    xol, xoh, xil, xih = _prescale_call(x_pad, dego, degi)
    tol1, toh1, til1, tih1 = _prop_kernel(xol, xoh, xil, xih, ei3, zrows)
    x1, xol1, xoh1, xil1, xih1 = _gate_mid_call(
        x_pad, tol1, toh1, til1, tih1, w0, bc0, dego, degi)
    tol2, toh2, til2, tih2 = _prop_kernel(xol1, xoh1, xil1, xih1, ei3, zrows)
    x2 = _gate_final_call(x1, tol2, toh2, til2, tih2, w1, bc1)
    return x2[:N]


# speedup vs baseline: 20.7386x; 1.5091x over previous
import functools

import jax
import jax.numpy as jnp
from jax import lax
from jax.experimental import pallas as pl
from jax.experimental.pallas import tpu as pltpu
from jax.experimental.pallas import tpu_sc as plsc

N = 10000
NP = 10240
E = 320000
D = 128
DH = D // 2

NUM_TILES = 16
EDGES_PER_TILE = E // NUM_TILES
CHUNK = 125
NCHUNK = EDGES_PER_TILE // CHUNK
ROWS_PER_TILE = NP // NUM_TILES
ZCHUNK = 128

RBLK = 1280

_mesh = plsc.VectorSubcoreMesh(core_axis_name="c", subcore_axis_name="s")
_sc_params = pltpu.CompilerParams(use_tc_tiling_on_sc=False)


@functools.partial(
    pl.kernel,
    out_type=jax.ShapeDtypeStruct((2, NP), jnp.float32),
    mesh=_mesh,
    scratch_types=[
        pltpu.VMEM((NCHUNK, CHUNK), jnp.int32),
        pltpu.VMEM((CHUNK,), jnp.float32),
        pltpu.VMEM_SHARED((NP,), jnp.float32),
        pltpu.SemaphoreType.DMA,
    ],
    compiler_params=_sc_params,
)
def _deg_kernel(ei3, zeros_rows, ones_hbm, deg2, idx, ones_v, dacc, sem):
    c = lax.axis_index("c")
    s = lax.axis_index("s")

    def run(r):
        pltpu.sync_copy(ones_hbm, ones_v)
        pltpu.sync_copy(zeros_rows, dacc.at[pl.ds(s * ROWS_PER_TILE, ROWS_PER_TILE)])
        plsc.subcore_barrier()
        pltpu.sync_copy(ei3.at[r, s], idx)

        def chunk(i, carry):
            pltpu.sync_copy(ones_v, dacc.at[idx.at[i]], add=True)
            return carry

        lax.fori_loop(0, NCHUNK, chunk, 0)
        plsc.subcore_barrier()
        pltpu.sync_copy(dacc.at[pl.ds(s * ROWS_PER_TILE, ROWS_PER_TILE)],
                        deg2.at[r, pl.ds(s * ROWS_PER_TILE, ROWS_PER_TILE)])

    @pl.when(c == 0)
    def _():
        run(0)

    @pl.when(c == 1)
    def _():
        run(1)


_half = jax.ShapeDtypeStruct((NP, DH), jnp.float32)


@functools.partial(
    pl.kernel,
    out_type=(_half, _half, _half, _half),
    mesh=_mesh,
    scratch_types=[
        pltpu.VMEM((NCHUNK, CHUNK), jnp.int32),
        pltpu.VMEM((NCHUNK, CHUNK), jnp.int32),
        pltpu.VMEM((CHUNK, DH), jnp.float32),
        pltpu.VMEM((CHUNK, DH), jnp.float32),
        pltpu.VMEM_SHARED((NP, DH), jnp.float32),
        pltpu.SemaphoreType.DMA,
        pltpu.SemaphoreType.DMA,
    ],
    compiler_params=_sc_params,
)
def _prop_kernel(xo_lo, xo_hi, xi_lo, xi_hi, ei3, zrows,
                 to_lo, to_hi, ti_lo, ti_hi,
                 idx_src, idx_dst, rows_a, rows_b, accum, sem_a, sem_b):
    c = lax.axis_index("c")
    s = lax.axis_index("s")

    def run(tables, src_row, dst_row, outs):
        pltpu.sync_copy(ei3.at[src_row, s], idx_src)
        pltpu.sync_copy(ei3.at[dst_row, s], idx_dst)
        for table, out in zip(tables, outs):
            for k in range(ROWS_PER_TILE // ZCHUNK):
                pltpu.sync_copy(
                    zrows, accum.at[pl.ds(s * ROWS_PER_TILE + k * ZCHUNK, ZCHUNK)])
            plsc.subcore_barrier()
            pltpu.async_copy(table.at[idx_src.at[0]], rows_a, sem_a)

            def pair(j, carry):
                i = 2 * j
                pltpu.make_async_copy(table.at[idx_src.at[i]], rows_a, sem_a).wait()
                pltpu.async_copy(table.at[idx_src.at[i + 1]], rows_b, sem_b)
                pltpu.sync_copy(rows_a, accum.at[idx_dst.at[i]], add=True)
                pltpu.make_async_copy(table.at[idx_src.at[i + 1]], rows_b, sem_b).wait()

                @pl.when(j < NCHUNK // 2 - 1)
                def _():
                    pltpu.async_copy(table.at[idx_src.at[i + 2]], rows_a, sem_a)

                pltpu.sync_copy(rows_b, accum.at[idx_dst.at[i + 1]], add=True)
                return carry

            lax.fori_loop(0, NCHUNK // 2, pair, 0)
            plsc.subcore_barrier()
            pltpu.sync_copy(accum.at[pl.ds(s * ROWS_PER_TILE, ROWS_PER_TILE)],
                            out.at[pl.ds(s * ROWS_PER_TILE, ROWS_PER_TILE)])

    @pl.when(c == 0)
    def _():
        run((xo_lo, xo_hi), 0, 1, (to_lo, to_hi))

    @pl.when(c == 1)
    def _():
        run((xi_lo, xi_hi), 1, 0, (ti_lo, ti_hi))


def _inv(d):
    return jnp.where(d > 0, 1.0 / d, 0.0)


def _prescale_body(x_ref, do_ref, di_ref, xolo_ref, xohi_ref, xilo_ref, xihi_ref):
    xv = x_ref[...]
    xo = xv * _inv(do_ref[...])
    xi = xv * _inv(di_ref[...])
    xolo_ref[...] = xo[:, :DH]
    xohi_ref[...] = xo[:, DH:]
    xilo_ref[...] = xi[:, :DH]
    xihi_ref[...] = xi[:, DH:]


def _gates(x_ref, tol_ref, toh_ref, til_ref, tih_ref, w_ref, b_ref):
    a = jnp.concatenate([x_ref[...], tol_ref[...], toh_ref[...],
                         til_ref[...], tih_ref[...]], axis=1)
    u = jnp.dot(a, w_ref[...], preferred_element_type=jnp.float32) + b_ref[...]
    z = jax.nn.sigmoid(u[:, :D])
    h = jnp.tanh(u[:, D:])
    return (1.0 - z) * h


def _gate_mid_body(x_ref, tol_ref, toh_ref, til_ref, tih_ref, w_ref, b_ref,
                   do_ref, di_ref,
                   xn_ref, xolo_ref, xohi_ref, xilo_ref, xihi_ref):
    xn = _gates(x_ref, tol_ref, toh_ref, til_ref, tih_ref, w_ref, b_ref)
    xn_ref[...] = xn
    xo = xn * _inv(do_ref[...])
    xi = xn * _inv(di_ref[...])
    xolo_ref[...] = xo[:, :DH]
    xohi_ref[...] = xo[:, DH:]
    xilo_ref[...] = xi[:, :DH]
    xihi_ref[...] = xi[:, DH:]


def _gate_final_body(x_ref, tol_ref, toh_ref, til_ref, tih_ref, w_ref, b_ref,
                     xn_ref):
    xn_ref[...] = _gates(x_ref, tol_ref, toh_ref, til_ref, tih_ref, w_ref, b_ref)


_rows_spec = pl.BlockSpec((RBLK, D), lambda i: (i, 0))
_hrows_spec = pl.BlockSpec((RBLK, DH), lambda i: (i, 0))
_col_spec = pl.BlockSpec((RBLK, 1), lambda i: (i, 0))
_w_spec = pl.BlockSpec((3 * D, 2 * D), lambda i: (0, 0))
_b_spec = pl.BlockSpec((1, 2 * D), lambda i: (0, 0))
_f32 = jnp.float32
_half_out = jax.ShapeDtypeStruct((NP, DH), _f32)
_full_out = jax.ShapeDtypeStruct((NP, D), _f32)

_prescale_call = pl.pallas_call(
    _prescale_body,
    grid=(NP // RBLK,),
    in_specs=[_rows_spec, _col_spec, _col_spec],
    out_specs=(_hrows_spec,) * 4,
    out_shape=(_half_out,) * 4,
)

_gate_mid_call = pl.pallas_call(
    _gate_mid_body,
    grid=(NP // RBLK,),
    in_specs=[_rows_spec] + [_hrows_spec] * 4 + [_w_spec, _b_spec,
                                                 _col_spec, _col_spec],
    out_specs=(_rows_spec,) + (_hrows_spec,) * 4,
    out_shape=(_full_out,) + (_half_out,) * 4,
)

_gate_final_call = pl.pallas_call(
    _gate_final_body,
    grid=(NP // RBLK,),
    in_specs=[_rows_spec] + [_hrows_spec] * 4 + [_w_spec, _b_spec],
    out_specs=_rows_spec,
    out_shape=_full_out,
)


def _pack_weights(Wz, bz, Wh, bh):
    wz = jnp.concatenate([Wz[0, 0, :D] + Wz[1, 0, :D], Wz[0, 1, :D], Wz[1, 1, :D]], axis=0)
    wh = jnp.concatenate([Wh[0, 0, :D] + Wh[1, 0, :D], Wh[0, 1, :D], Wh[1, 1, :D]], axis=0)
    w = jnp.concatenate([wz, wh], axis=1)
    b = jnp.concatenate([bz, bh])[None, :]
    return w, b


def kernel(x, edge_index, W0_z, b0_z, W0_r, b0_r, W0_h, b0_h,
           W1_z, b1_z, W1_r, b1_r, W1_h, b1_h):
    ei3 = edge_index.astype(jnp.int32).reshape(2, NUM_TILES, NCHUNK, CHUNK)
    x_pad = jnp.zeros((NP, D), jnp.float32).at[:N].set(x)
    w0, bc0 = _pack_weights(W0_z, b0_z, W0_h, b0_h)
    w1, bc1 = _pack_weights(W1_z, b1_z, W1_h, b1_h)

    zeros_rows = jnp.zeros((ROWS_PER_TILE,), jnp.float32)
    ones_hbm = jnp.ones((CHUNK,), jnp.float32)
    zrows = jnp.zeros((ZCHUNK, DH), jnp.float32)

    deg2 = _deg_kernel(ei3, zeros_rows, ones_hbm)
    dego = deg2[0].reshape(NP, 1)
    degi = deg2[1].reshape(NP, 1)

    xol, xoh, xil, xih = _prescale_call(x_pad, dego, degi)
    tol1, toh1, til1, tih1 = _prop_kernel(xol, xoh, xil, xih, ei3, zrows)
    x1, xol1, xoh1, xil1, xih1 = _gate_mid_call(
        x_pad, tol1, toh1, til1, tih1, w0, bc0, dego, degi)
    tol2, toh2, til2, tih2 = _prop_kernel(xol1, xoh1, xil1, xih1, ei3, zrows)
    x2 = _gate_final_call(x1, tol2, toh2, til2, tih2, w1, bc1)
    return x2[:N]


# speedup vs baseline: 28.3813x; 1.3685x over previous
import functools

import jax
import jax.numpy as jnp
from jax import lax
from jax.experimental import pallas as pl
from jax.experimental.pallas import tpu as pltpu
from jax.experimental.pallas import tpu_sc as plsc

N = 10000
NP = 10240
E = 320000
D = 128

HALVES = 2
DH = D // HALVES
SLOTS = 4

NUM_TILES = 16
EDGES_PER_TILE = E // NUM_TILES
CHUNK = 125
NCHUNK = EDGES_PER_TILE // CHUNK
ROWS_PER_TILE = NP // NUM_TILES
ZCHUNK = 128

RBLK = 1280

_mesh = plsc.VectorSubcoreMesh(core_axis_name="c", subcore_axis_name="s")
_sc_params = pltpu.CompilerParams(use_tc_tiling_on_sc=False)


@functools.partial(
    pl.kernel,
    out_type=jax.ShapeDtypeStruct((2, NP), jnp.float32),
    mesh=_mesh,
    scratch_types=[
        pltpu.VMEM((NCHUNK, CHUNK), jnp.int32),
        pltpu.VMEM((CHUNK,), jnp.float32),
        pltpu.VMEM_SHARED((NP,), jnp.float32),
        pltpu.SemaphoreType.DMA,
    ],
    compiler_params=_sc_params,
)
def _deg_kernel(ei3, zeros_rows, ones_hbm, deg2, idx, ones_v, dacc, sem):
    c = lax.axis_index("c")
    s = lax.axis_index("s")

    def run(r):
        pltpu.sync_copy(ones_hbm, ones_v)
        pltpu.sync_copy(zeros_rows, dacc.at[pl.ds(s * ROWS_PER_TILE, ROWS_PER_TILE)])
        plsc.subcore_barrier()
        pltpu.sync_copy(ei3.at[r, s], idx)

        def chunk(i, carry):
            pltpu.sync_copy(ones_v, dacc.at[idx.at[i]], add=True)
            return carry

        lax.fori_loop(0, NCHUNK, chunk, 0)
        plsc.subcore_barrier()
        pltpu.sync_copy(dacc.at[pl.ds(s * ROWS_PER_TILE, ROWS_PER_TILE)],
                        deg2.at[r, pl.ds(s * ROWS_PER_TILE, ROWS_PER_TILE)])

    @pl.when(c == 0)
    def _():
        run(0)

    @pl.when(c == 1)
    def _():
        run(1)


_half = jax.ShapeDtypeStruct((NP, DH), jnp.float32)


@functools.partial(
    pl.kernel,
    out_type=(_half,) * (2 * HALVES),
    mesh=_mesh,
    scratch_types=[
        pltpu.VMEM((NCHUNK, CHUNK), jnp.int32),
        pltpu.VMEM((NCHUNK, CHUNK), jnp.int32),
        pltpu.VMEM((SLOTS, CHUNK, DH), jnp.float32),
        pltpu.VMEM_SHARED((NP, DH), jnp.float32),
    ] + [pltpu.SemaphoreType.DMA] * (2 * SLOTS),
    compiler_params=_sc_params,
)
def _prop_kernel(*refs):
    xo = refs[0:HALVES]
    xi = refs[HALVES:2 * HALVES]
    ei3, zrows = refs[2 * HALVES], refs[2 * HALVES + 1]
    to = refs[2 * HALVES + 2:3 * HALVES + 2]
    ti = refs[3 * HALVES + 2:4 * HALVES + 2]
    idx_src, idx_dst, rows, accum = refs[4 * HALVES + 2:4 * HALVES + 6]
    gsem = refs[4 * HALVES + 6:4 * HALVES + 6 + SLOTS]
    ssem = refs[4 * HALVES + 6 + SLOTS:]
    c = lax.axis_index("c")
    s = lax.axis_index("s")

    def run(tables, src_row, dst_row, outs):
        pltpu.sync_copy(ei3.at[src_row, s], idx_src)
        pltpu.sync_copy(ei3.at[dst_row, s], idx_dst)
        for table, out in zip(tables, outs):
            for k in range(ROWS_PER_TILE // ZCHUNK):
                pltpu.sync_copy(
                    zrows, accum.at[pl.ds(s * ROWS_PER_TILE + k * ZCHUNK, ZCHUNK)])
            plsc.subcore_barrier()
            for k in range(SLOTS):
                pltpu.async_copy(table.at[idx_src.at[k]], rows.at[k], gsem[k])

            def group(j, carry):
                base = SLOTS * j
                for k in range(SLOTS):
                    i = base + k
                    pltpu.make_async_copy(
                        table.at[idx_src.at[i]], rows.at[k], gsem[k]).wait()
                    pltpu.async_copy(
                        rows.at[k], accum.at[idx_dst.at[i]], ssem[k], add=True)
                for k in range(SLOTS):
                    i = base + k

                    @pl.when(i + SLOTS < NCHUNK)
                    def _():
                        pltpu.make_async_copy(
                            rows.at[k], accum.at[idx_dst.at[i]], ssem[k]).wait()
                        pltpu.async_copy(
                            table.at[idx_src.at[i + SLOTS]], rows.at[k], gsem[k])

                return carry

            lax.fori_loop(0, NCHUNK // SLOTS, group, 0)
            for k in range(SLOTS):
                i = NCHUNK - SLOTS + k
                pltpu.make_async_copy(
                    rows.at[k], accum.at[idx_dst.at[i]], ssem[k]).wait()
            plsc.subcore_barrier()
            pltpu.sync_copy(accum.at[pl.ds(s * ROWS_PER_TILE, ROWS_PER_TILE)],
                            out.at[pl.ds(s * ROWS_PER_TILE, ROWS_PER_TILE)])

    @pl.when(c == 0)
    def _():
        run(xo, 0, 1, to)

    @pl.when(c == 1)
    def _():
        run(xi, 1, 0, ti)


def _inv(d):
    return jnp.where(d > 0, 1.0 / d, 0.0)


def _split_store(refs, val):
    for h, ref in enumerate(refs):
        ref[...] = val[:, h * DH:(h + 1) * DH]


def _prescale_body(x_ref, do_ref, di_ref, *out_refs):
    xv = x_ref[...]
    _split_store(out_refs[:HALVES], xv * _inv(do_ref[...]))
    _split_store(out_refs[HALVES:], xv * _inv(di_ref[...]))


def _gates(x_ref, t_refs, w_ref, b_ref):
    a = jnp.concatenate([x_ref[...]] + [t[...] for t in t_refs], axis=1)
    u = jnp.dot(a, w_ref[...], preferred_element_type=jnp.float32) + b_ref[...]
    z = jax.nn.sigmoid(u[:, :D])
    h = jnp.tanh(u[:, D:])
    return (1.0 - z) * h


def _gate_mid_body(*refs):
    x_ref = refs[0]
    t_refs = refs[1:2 * HALVES + 1]
    w_ref, b_ref, do_ref, di_ref = refs[2 * HALVES + 1:2 * HALVES + 5]
    xn_ref = refs[2 * HALVES + 5]
    out_refs = refs[2 * HALVES + 6:]
    xn = _gates(x_ref, t_refs, w_ref, b_ref)
    xn_ref[...] = xn
    _split_store(out_refs[:HALVES], xn * _inv(do_ref[...]))
    _split_store(out_refs[HALVES:], xn * _inv(di_ref[...]))


def _gate_final_body(*refs):
    x_ref = refs[0]
    t_refs = refs[1:2 * HALVES + 1]
    w_ref, b_ref, xn_ref = refs[2 * HALVES + 1:]
    xn_ref[...] = _gates(x_ref, t_refs, w_ref, b_ref)


_rows_spec = pl.BlockSpec((RBLK, D), lambda i: (i, 0))
_hrows_spec = pl.BlockSpec((RBLK, DH), lambda i: (i, 0))
_col_spec = pl.BlockSpec((RBLK, 1), lambda i: (i, 0))
_w_spec = pl.BlockSpec((3 * D, 2 * D), lambda i: (0, 0))
_b_spec = pl.BlockSpec((1, 2 * D), lambda i: (0, 0))
_f32 = jnp.float32
_half_out = jax.ShapeDtypeStruct((NP, DH), _f32)
_full_out = jax.ShapeDtypeStruct((NP, D), _f32)

_prescale_call = pl.pallas_call(
    _prescale_body,
    grid=(NP // RBLK,),
    in_specs=[_rows_spec, _col_spec, _col_spec],
    out_specs=(_hrows_spec,) * (2 * HALVES),
    out_shape=(_half_out,) * (2 * HALVES),
)

_gate_mid_call = pl.pallas_call(
    _gate_mid_body,
    grid=(NP // RBLK,),
    in_specs=[_rows_spec] + [_hrows_spec] * (2 * HALVES)
             + [_w_spec, _b_spec, _col_spec, _col_spec],
    out_specs=(_rows_spec,) + (_hrows_spec,) * (2 * HALVES),
    out_shape=(_full_out,) + (_half_out,) * (2 * HALVES),
)

_gate_final_call = pl.pallas_call(
    _gate_final_body,
    grid=(NP // RBLK,),
    in_specs=[_rows_spec] + [_hrows_spec] * (2 * HALVES) + [_w_spec, _b_spec],
    out_specs=_rows_spec,
    out_shape=_full_out,
)


def _pack_weights(Wz, bz, Wh, bh):
    wz = jnp.concatenate([Wz[0, 0, :D] + Wz[1, 0, :D], Wz[0, 1, :D], Wz[1, 1, :D]], axis=0)
    wh = jnp.concatenate([Wh[0, 0, :D] + Wh[1, 0, :D], Wh[0, 1, :D], Wh[1, 1, :D]], axis=0)
    w = jnp.concatenate([wz, wh], axis=1)
    b = jnp.concatenate([bz, bh])[None, :]
    return w, b


def kernel(x, edge_index, W0_z, b0_z, W0_r, b0_r, W0_h, b0_h,
           W1_z, b1_z, W1_r, b1_r, W1_h, b1_h):
    ei3 = edge_index.astype(jnp.int32).reshape(2, NUM_TILES, NCHUNK, CHUNK)
    x_pad = jnp.zeros((NP, D), jnp.float32).at[:N].set(x)
    w0, bc0 = _pack_weights(W0_z, b0_z, W0_h, b0_h)
    w1, bc1 = _pack_weights(W1_z, b1_z, W1_h, b1_h)

    zeros_rows = jnp.zeros((ROWS_PER_TILE,), jnp.float32)
    ones_hbm = jnp.ones((CHUNK,), jnp.float32)
    zrows = jnp.zeros((ZCHUNK, DH), jnp.float32)

    deg2 = _deg_kernel(ei3, zeros_rows, ones_hbm)
    dego = deg2[0].reshape(NP, 1)
    degi = deg2[1].reshape(NP, 1)

    xs1 = _prescale_call(x_pad, dego, degi)
    t1 = _prop_kernel(*xs1, ei3, zrows)
    x1, *xs2 = _gate_mid_call(x_pad, *t1, w0, bc0, dego, degi)
    t2 = _prop_kernel(*xs2, ei3, zrows)
    x2 = _gate_final_call(x1, *t2, w1, bc1)
    return x2[:N]


# speedup vs baseline: 28.8129x; 1.0152x over previous
import functools

import jax
import jax.numpy as jnp
from jax import lax
from jax.experimental import pallas as pl
from jax.experimental.pallas import tpu as pltpu
from jax.experimental.pallas import tpu_sc as plsc

N = 10000
NP = 10240
E = 320000
D = 128

HALVES = 2
DH = D // HALVES
SLOTS = 5

NUM_TILES = 16
EDGES_PER_TILE = E // NUM_TILES
CHUNK = 125
NCHUNK = EDGES_PER_TILE // CHUNK
ROWS_PER_TILE = NP // NUM_TILES
ZCHUNK = 128

RBLK = 1280

_mesh = plsc.VectorSubcoreMesh(core_axis_name="c", subcore_axis_name="s")
_sc_params = pltpu.CompilerParams(use_tc_tiling_on_sc=False)


@functools.partial(
    pl.kernel,
    out_type=jax.ShapeDtypeStruct((2, NP), jnp.float32),
    mesh=_mesh,
    scratch_types=[
        pltpu.VMEM((NCHUNK, CHUNK), jnp.int32),
        pltpu.VMEM((CHUNK,), jnp.float32),
        pltpu.VMEM_SHARED((NP,), jnp.float32),
        pltpu.SemaphoreType.DMA,
    ],
    compiler_params=_sc_params,
)
def _deg_kernel(ei3, zeros_rows, ones_hbm, deg2, idx, ones_v, dacc, sem):
    c = lax.axis_index("c")
    s = lax.axis_index("s")

    def run(r):
        pltpu.sync_copy(ones_hbm, ones_v)
        pltpu.sync_copy(zeros_rows, dacc.at[pl.ds(s * ROWS_PER_TILE, ROWS_PER_TILE)])
        plsc.subcore_barrier()
        pltpu.sync_copy(ei3.at[r, s], idx)

        def chunk(i, carry):
            pltpu.sync_copy(ones_v, dacc.at[idx.at[i]], add=True)
            return carry

        lax.fori_loop(0, NCHUNK, chunk, 0)
        plsc.subcore_barrier()
        pltpu.sync_copy(dacc.at[pl.ds(s * ROWS_PER_TILE, ROWS_PER_TILE)],
                        deg2.at[r, pl.ds(s * ROWS_PER_TILE, ROWS_PER_TILE)])

    @pl.when(c == 0)
    def _():
        run(0)

    @pl.when(c == 1)
    def _():
        run(1)


_half = jax.ShapeDtypeStruct((NP, DH), jnp.float32)


@functools.partial(
    pl.kernel,
    out_type=(_half,) * (2 * HALVES),
    mesh=_mesh,
    scratch_types=[
        pltpu.VMEM((NCHUNK, CHUNK), jnp.int32),
        pltpu.VMEM((NCHUNK, CHUNK), jnp.int32),
        pltpu.VMEM((SLOTS, CHUNK, DH), jnp.float32),
        pltpu.VMEM_SHARED((NP, DH), jnp.float32),
    ] + [pltpu.SemaphoreType.DMA] * (2 * SLOTS),
    compiler_params=_sc_params,
)
def _prop_kernel(*refs):
    xo = refs[0:HALVES]
    xi = refs[HALVES:2 * HALVES]
    ei3, zrows = refs[2 * HALVES], refs[2 * HALVES + 1]
    to = refs[2 * HALVES + 2:3 * HALVES + 2]
    ti = refs[3 * HALVES + 2:4 * HALVES + 2]
    idx_src, idx_dst, rows, accum = refs[4 * HALVES + 2:4 * HALVES + 6]
    gsem = refs[4 * HALVES + 6:4 * HALVES + 6 + SLOTS]
    ssem = refs[4 * HALVES + 6 + SLOTS:]
    c = lax.axis_index("c")
    s = lax.axis_index("s")

    def run(tables, src_row, dst_row, outs):
        pltpu.sync_copy(ei3.at[src_row, s], idx_src)
        pltpu.sync_copy(ei3.at[dst_row, s], idx_dst)
        for table, out in zip(tables, outs):
            for k in range(ROWS_PER_TILE // ZCHUNK):
                pltpu.sync_copy(
                    zrows, accum.at[pl.ds(s * ROWS_PER_TILE + k * ZCHUNK, ZCHUNK)])
            plsc.subcore_barrier()
            for k in range(SLOTS):
                pltpu.async_copy(table.at[idx_src.at[k]], rows.at[k], gsem[k])

            def group(j, carry):
                base = SLOTS * j
                for k in range(SLOTS):
                    i = base + k
                    pltpu.make_async_copy(
                        table.at[idx_src.at[i]], rows.at[k], gsem[k]).wait()
                    pltpu.async_copy(
                        rows.at[k], accum.at[idx_dst.at[i]], ssem[k], add=True)
                for k in range(SLOTS):
                    i = base + k

                    @pl.when(i + SLOTS < NCHUNK)
                    def _():
                        pltpu.make_async_copy(
                            rows.at[k], accum.at[idx_dst.at[i]], ssem[k]).wait()
                        pltpu.async_copy(
                            table.at[idx_src.at[i + SLOTS]], rows.at[k], gsem[k])

                return carry

            lax.fori_loop(0, NCHUNK // SLOTS, group, 0)
            for k in range(SLOTS):
                i = NCHUNK - SLOTS + k
                pltpu.make_async_copy(
                    rows.at[k], accum.at[idx_dst.at[i]], ssem[k]).wait()
            plsc.subcore_barrier()
            pltpu.sync_copy(accum.at[pl.ds(s * ROWS_PER_TILE, ROWS_PER_TILE)],
                            out.at[pl.ds(s * ROWS_PER_TILE, ROWS_PER_TILE)])

    @pl.when(c == 0)
    def _():
        run(xo, 0, 1, to)

    @pl.when(c == 1)
    def _():
        run(xi, 1, 0, ti)


def _inv(d):
    return jnp.where(d > 0, 1.0 / d, 0.0)


def _split_store(refs, val):
    for h, ref in enumerate(refs):
        ref[...] = val[:, h * DH:(h + 1) * DH]


def _prescale_body(x_ref, do_ref, di_ref, *out_refs):
    xv = x_ref[...]
    _split_store(out_refs[:HALVES], xv * _inv(do_ref[...]))
    _split_store(out_refs[HALVES:], xv * _inv(di_ref[...]))


def _gates(x_ref, t_refs, w_ref, b_ref):
    a = jnp.concatenate([x_ref[...]] + [t[...] for t in t_refs], axis=1)
    u = jnp.dot(a, w_ref[...], preferred_element_type=jnp.float32) + b_ref[...]
    z = jax.nn.sigmoid(u[:, :D])
    h = jnp.tanh(u[:, D:])
    return (1.0 - z) * h


def _gate_mid_body(*refs):
    x_ref = refs[0]
    t_refs = refs[1:2 * HALVES + 1]
    w_ref, b_ref, do_ref, di_ref = refs[2 * HALVES + 1:2 * HALVES + 5]
    xn_ref = refs[2 * HALVES + 5]
    out_refs = refs[2 * HALVES + 6:]
    xn = _gates(x_ref, t_refs, w_ref, b_ref)
    xn_ref[...] = xn
    _split_store(out_refs[:HALVES], xn * _inv(do_ref[...]))
    _split_store(out_refs[HALVES:], xn * _inv(di_ref[...]))


def _gate_final_body(*refs):
    x_ref = refs[0]
    t_refs = refs[1:2 * HALVES + 1]
    w_ref, b_ref, xn_ref = refs[2 * HALVES + 1:]
    xn_ref[...] = _gates(x_ref, t_refs, w_ref, b_ref)


_rows_spec = pl.BlockSpec((RBLK, D), lambda i: (i, 0))
_hrows_spec = pl.BlockSpec((RBLK, DH), lambda i: (i, 0))
_col_spec = pl.BlockSpec((RBLK, 1), lambda i: (i, 0))
_w_spec = pl.BlockSpec((3 * D, 2 * D), lambda i: (0, 0))
_b_spec = pl.BlockSpec((1, 2 * D), lambda i: (0, 0))
_f32 = jnp.float32
_half_out = jax.ShapeDtypeStruct((NP, DH), _f32)
_full_out = jax.ShapeDtypeStruct((NP, D), _f32)

_prescale_call = pl.pallas_call(
    _prescale_body,
    grid=(NP // RBLK,),
    in_specs=[_rows_spec, _col_spec, _col_spec],
    out_specs=(_hrows_spec,) * (2 * HALVES),
    out_shape=(_half_out,) * (2 * HALVES),
)

_gate_mid_call = pl.pallas_call(
    _gate_mid_body,
    grid=(NP // RBLK,),
    in_specs=[_rows_spec] + [_hrows_spec] * (2 * HALVES)
             + [_w_spec, _b_spec, _col_spec, _col_spec],
    out_specs=(_rows_spec,) + (_hrows_spec,) * (2 * HALVES),
    out_shape=(_full_out,) + (_half_out,) * (2 * HALVES),
)

_gate_final_call = pl.pallas_call(
    _gate_final_body,
    grid=(NP // RBLK,),
    in_specs=[_rows_spec] + [_hrows_spec] * (2 * HALVES) + [_w_spec, _b_spec],
    out_specs=_rows_spec,
    out_shape=_full_out,
)


def _pack_weights(Wz, bz, Wh, bh):
    wz = jnp.concatenate([Wz[0, 0, :D] + Wz[1, 0, :D], Wz[0, 1, :D], Wz[1, 1, :D]], axis=0)
    wh = jnp.concatenate([Wh[0, 0, :D] + Wh[1, 0, :D], Wh[0, 1, :D], Wh[1, 1, :D]], axis=0)
    w = jnp.concatenate([wz, wh], axis=1)
    b = jnp.concatenate([bz, bh])[None, :]
    return w, b


def kernel(x, edge_index, W0_z, b0_z, W0_r, b0_r, W0_h, b0_h,
           W1_z, b1_z, W1_r, b1_r, W1_h, b1_h):
    ei3 = edge_index.astype(jnp.int32).reshape(2, NUM_TILES, NCHUNK, CHUNK)
    x_pad = jnp.zeros((NP, D), jnp.float32).at[:N].set(x)
    w0, bc0 = _pack_weights(W0_z, b0_z, W0_h, b0_h)
    w1, bc1 = _pack_weights(W1_z, b1_z, W1_h, b1_h)

    zeros_rows = jnp.zeros((ROWS_PER_TILE,), jnp.float32)
    ones_hbm = jnp.ones((CHUNK,), jnp.float32)
    zrows = jnp.zeros((ZCHUNK, DH), jnp.float32)

    deg2 = _deg_kernel(ei3, zeros_rows, ones_hbm)
    dego = deg2[0].reshape(NP, 1)
    degi = deg2[1].reshape(NP, 1)

    xs1 = _prescale_call(x_pad, dego, degi)
    t1 = _prop_kernel(*xs1, ei3, zrows)
    x1, *xs2 = _gate_mid_call(x_pad, *t1, w0, bc0, dego, degi)
    t2 = _prop_kernel(*xs2, ei3, zrows)
    x2 = _gate_final_call(x1, *t2, w1, bc1)
    return x2[:N]


# speedup vs baseline: 29.1911x; 1.0131x over previous
import functools

import jax
import jax.numpy as jnp
from jax import lax
from jax.experimental import pallas as pl
from jax.experimental.pallas import tpu as pltpu
from jax.experimental.pallas import tpu_sc as plsc

N = 10000
NP = 10240
E = 320000
D = 128

HALVES = 2
DH = D // HALVES
SLOTS = 5

NUM_TILES = 16
EDGES_PER_TILE = E // NUM_TILES
CHUNK = 125
NCHUNK = EDGES_PER_TILE // CHUNK
ROWS_PER_TILE = NP // NUM_TILES
ZCHUNK = 128

RBLK = 2000

_mesh = plsc.VectorSubcoreMesh(core_axis_name="c", subcore_axis_name="s")
_sc_params = pltpu.CompilerParams(use_tc_tiling_on_sc=False)


_col = jax.ShapeDtypeStruct((NP, 1), jnp.float32)


@functools.partial(
    pl.kernel,
    out_type=(_col, _col),
    mesh=_mesh,
    scratch_types=[
        pltpu.VMEM((NCHUNK, CHUNK), jnp.int32),
        pltpu.VMEM((CHUNK, 1), jnp.float32),
        pltpu.VMEM_SHARED((NP, 1), jnp.float32),
        pltpu.SemaphoreType.DMA,
    ],
    compiler_params=_sc_params,
)
def _deg_kernel(ei3, zeros_col, ones_hbm, dego, degi, idx, ones_v, dacc, sem):
    c = lax.axis_index("c")
    s = lax.axis_index("s")

    def run(r, out):
        pltpu.sync_copy(ones_hbm, ones_v)
        pltpu.sync_copy(zeros_col, dacc.at[pl.ds(s * ROWS_PER_TILE, ROWS_PER_TILE)])
        plsc.subcore_barrier()
        pltpu.sync_copy(ei3.at[r, s], idx)

        def chunk(i, carry):
            pltpu.sync_copy(ones_v, dacc.at[idx.at[i]], add=True)
            return carry

        lax.fori_loop(0, NCHUNK, chunk, 0)
        plsc.subcore_barrier()
        pltpu.sync_copy(dacc.at[pl.ds(s * ROWS_PER_TILE, ROWS_PER_TILE)],
                        out.at[pl.ds(s * ROWS_PER_TILE, ROWS_PER_TILE)])

    @pl.when(c == 0)
    def _():
        run(0, dego)

    @pl.when(c == 1)
    def _():
        run(1, degi)


_half = jax.ShapeDtypeStruct((NP, DH), jnp.float32)


@functools.partial(
    pl.kernel,
    out_type=(_half,) * (2 * HALVES),
    mesh=_mesh,
    scratch_types=[
        pltpu.VMEM((NCHUNK, CHUNK), jnp.int32),
        pltpu.VMEM((NCHUNK, CHUNK), jnp.int32),
        pltpu.VMEM((SLOTS, CHUNK, DH), jnp.float32),
        pltpu.VMEM_SHARED((NP, DH), jnp.float32),
    ] + [pltpu.SemaphoreType.DMA] * (2 * SLOTS),
    compiler_params=_sc_params,
)
def _prop_kernel(*refs):
    xo = refs[0:HALVES]
    xi = refs[HALVES:2 * HALVES]
    ei3, zrows = refs[2 * HALVES], refs[2 * HALVES + 1]
    to = refs[2 * HALVES + 2:3 * HALVES + 2]
    ti = refs[3 * HALVES + 2:4 * HALVES + 2]
    idx_src, idx_dst, rows, accum = refs[4 * HALVES + 2:4 * HALVES + 6]
    gsem = refs[4 * HALVES + 6:4 * HALVES + 6 + SLOTS]
    ssem = refs[4 * HALVES + 6 + SLOTS:]
    c = lax.axis_index("c")
    s = lax.axis_index("s")

    def run(tables, src_row, dst_row, outs):
        pltpu.sync_copy(ei3.at[src_row, s], idx_src)
        pltpu.sync_copy(ei3.at[dst_row, s], idx_dst)
        for table, out in zip(tables, outs):
            for k in range(ROWS_PER_TILE // ZCHUNK):
                pltpu.sync_copy(
                    zrows, accum.at[pl.ds(s * ROWS_PER_TILE + k * ZCHUNK, ZCHUNK)])
            plsc.subcore_barrier()
            for k in range(SLOTS):
                pltpu.async_copy(table.at[idx_src.at[k]], rows.at[k], gsem[k])

            def group(j, carry):
                base = SLOTS * j
                for k in range(SLOTS):
                    i = base + k
                    pltpu.make_async_copy(
                        table.at[idx_src.at[i]], rows.at[k], gsem[k]).wait()
                    pltpu.async_copy(
                        rows.at[k], accum.at[idx_dst.at[i]], ssem[k], add=True)
                for k in range(SLOTS):
                    i = base + k

                    @pl.when(i + SLOTS < NCHUNK)
                    def _():
                        pltpu.make_async_copy(
                            rows.at[k], accum.at[idx_dst.at[i]], ssem[k]).wait()
                        pltpu.async_copy(
                            table.at[idx_src.at[i + SLOTS]], rows.at[k], gsem[k])

                return carry

            lax.fori_loop(0, NCHUNK // SLOTS, group, 0)
            for k in range(SLOTS):
                i = NCHUNK - SLOTS + k
                pltpu.make_async_copy(
                    rows.at[k], accum.at[idx_dst.at[i]], ssem[k]).wait()
            plsc.subcore_barrier()
            pltpu.sync_copy(accum.at[pl.ds(s * ROWS_PER_TILE, ROWS_PER_TILE)],
                            out.at[pl.ds(s * ROWS_PER_TILE, ROWS_PER_TILE)])

    @pl.when(c == 0)
    def _():
        run(xo, 0, 1, to)

    @pl.when(c == 1)
    def _():
        run(xi, 1, 0, ti)


def _inv(d):
    return jnp.where(d > 0, 1.0 / d, 0.0)


def _split_store(refs, val):
    for h, ref in enumerate(refs):
        ref[...] = val[:, h * DH:(h + 1) * DH]


def _prescale_body(x_ref, do_ref, di_ref, *out_refs):
    xv = x_ref[...]
    _split_store(out_refs[:HALVES], xv * _inv(do_ref[...]))
    _split_store(out_refs[HALVES:], xv * _inv(di_ref[...]))


def _gates(x_ref, t_refs, w_ref, b_ref):
    a = jnp.concatenate([x_ref[...]] + [t[...] for t in t_refs], axis=1)
    u = jnp.dot(a, w_ref[...], preferred_element_type=jnp.float32) + b_ref[...]
    z = jax.nn.sigmoid(u[:, :D])
    h = jnp.tanh(u[:, D:])
    return (1.0 - z) * h


def _gate_mid_body(*refs):
    x_ref = refs[0]
    t_refs = refs[1:2 * HALVES + 1]
    w_ref, b_ref, do_ref, di_ref = refs[2 * HALVES + 1:2 * HALVES + 5]
    xn_ref = refs[2 * HALVES + 5]
    out_refs = refs[2 * HALVES + 6:]
    xn = _gates(x_ref, t_refs, w_ref, b_ref)
    xn_ref[...] = xn
    _split_store(out_refs[:HALVES], xn * _inv(do_ref[...]))
    _split_store(out_refs[HALVES:], xn * _inv(di_ref[...]))


def _gate_final_body(*refs):
    x_ref = refs[0]
    t_refs = refs[1:2 * HALVES + 1]
    w_ref, b_ref, xn_ref = refs[2 * HALVES + 1:]
    xn_ref[...] = _gates(x_ref, t_refs, w_ref, b_ref)


_rows_spec = pl.BlockSpec((RBLK, D), lambda i: (i, 0))
_hrows_spec = pl.BlockSpec((RBLK, DH), lambda i: (i, 0))
_col_spec = pl.BlockSpec((RBLK, 1), lambda i: (i, 0))
_w_spec = pl.BlockSpec((3 * D, 2 * D), lambda i: (0, 0))
_b_spec = pl.BlockSpec((1, 2 * D), lambda i: (0, 0))
_f32 = jnp.float32
_half_out = jax.ShapeDtypeStruct((N, DH), _f32)
_full_out = jax.ShapeDtypeStruct((N, D), _f32)

_prescale_call = pl.pallas_call(
    _prescale_body,
    grid=(N // RBLK,),
    in_specs=[_rows_spec, _col_spec, _col_spec],
    out_specs=(_hrows_spec,) * (2 * HALVES),
    out_shape=(_half_out,) * (2 * HALVES),
)

_gate_mid_call = pl.pallas_call(
    _gate_mid_body,
    grid=(N // RBLK,),
    in_specs=[_rows_spec] + [_hrows_spec] * (2 * HALVES)
             + [_w_spec, _b_spec, _col_spec, _col_spec],
    out_specs=(_rows_spec,) + (_hrows_spec,) * (2 * HALVES),
    out_shape=(_full_out,) + (_half_out,) * (2 * HALVES),
)

_gate_final_call = pl.pallas_call(
    _gate_final_body,
    grid=(N // RBLK,),
    in_specs=[_rows_spec] + [_hrows_spec] * (2 * HALVES) + [_w_spec, _b_spec],
    out_specs=_rows_spec,
    out_shape=_full_out,
)


def _pack_weights(Wz, bz, Wh, bh):
    wz = jnp.concatenate([Wz[0, 0, :D] + Wz[1, 0, :D], Wz[0, 1, :D], Wz[1, 1, :D]], axis=0)
    wh = jnp.concatenate([Wh[0, 0, :D] + Wh[1, 0, :D], Wh[0, 1, :D], Wh[1, 1, :D]], axis=0)
    w = jnp.concatenate([wz, wh], axis=1)
    b = jnp.concatenate([bz, bh])[None, :]
    return w, b


def kernel(x, edge_index, W0_z, b0_z, W0_r, b0_r, W0_h, b0_h,
           W1_z, b1_z, W1_r, b1_r, W1_h, b1_h):
    ei3 = edge_index.astype(jnp.int32).reshape(2, NUM_TILES, NCHUNK, CHUNK)
    w0, bc0 = _pack_weights(W0_z, b0_z, W0_h, b0_h)
    w1, bc1 = _pack_weights(W1_z, b1_z, W1_h, b1_h)

    zeros_col = jnp.zeros((ROWS_PER_TILE, 1), jnp.float32)
    ones_col = jnp.ones((CHUNK, 1), jnp.float32)
    zrows = jnp.zeros((ZCHUNK, DH), jnp.float32)

    dego, degi = _deg_kernel(ei3, zeros_col, ones_col)

    xs1 = _prescale_call(x, dego, degi)
    t1 = _prop_kernel(*xs1, ei3, zrows)
    x1, *xs2 = _gate_mid_call(x, *t1, w0, bc0, dego, degi)
    t2 = _prop_kernel(*xs2, ei3, zrows)
    x2 = _gate_final_call(x1, *t2, w1, bc1)
    return x2


# speedup vs baseline: 30.1929x; 1.0343x over previous
import functools

import jax
import jax.numpy as jnp
from jax import lax
from jax.experimental import pallas as pl
from jax.experimental.pallas import tpu as pltpu
from jax.experimental.pallas import tpu_sc as plsc

N = 10000
NP = 10240
E = 320000
D = 128

HALVES = 2
DH = D // HALVES
SLOTS = 5

NUM_TILES = 16
EDGES_PER_TILE = E // NUM_TILES
CHUNK = 125
NCHUNK = EDGES_PER_TILE // CHUNK
ROWS_PER_TILE = NP // NUM_TILES
ZCHUNK = 128

RBLK = 2000

_mesh = plsc.VectorSubcoreMesh(core_axis_name="c", subcore_axis_name="s")
_sc_params = pltpu.CompilerParams(use_tc_tiling_on_sc=False)


@functools.partial(
    pl.kernel,
    out_type=jax.ShapeDtypeStruct((2, NP), jnp.float32),
    mesh=_mesh,
    scratch_types=[
        pltpu.VMEM((NCHUNK, CHUNK), jnp.int32),
        pltpu.VMEM((CHUNK,), jnp.float32),
        pltpu.VMEM_SHARED((NP,), jnp.float32),
        pltpu.SemaphoreType.DMA,
    ],
    compiler_params=_sc_params,
)
def _deg_kernel(ei3, zeros_col, ones_hbm, deg2, idx, ones_v, dacc, sem):
    c = lax.axis_index("c")
    s = lax.axis_index("s")

    def run(r, out):
        pltpu.sync_copy(ones_hbm, ones_v)
        pltpu.sync_copy(zeros_col, dacc.at[pl.ds(s * ROWS_PER_TILE, ROWS_PER_TILE)])
        pltpu.sync_copy(ei3.at[r, s], idx)
        plsc.subcore_barrier()
        for k in range(8):
            pltpu.async_copy(ones_v, dacc.at[idx.at[k]], sem, add=True)

        def chunk(i, carry):
            pltpu.make_async_copy(ones_v, dacc.at[idx.at[i]], sem).wait()

            @pl.when(i + 8 < NCHUNK)
            def _():
                pltpu.async_copy(ones_v, dacc.at[idx.at[i + 8]], sem, add=True)

            return carry

        lax.fori_loop(0, NCHUNK, chunk, 0)
        plsc.subcore_barrier()
        pltpu.sync_copy(dacc.at[pl.ds(s * ROWS_PER_TILE, ROWS_PER_TILE)],
                        deg2.at[out, pl.ds(s * ROWS_PER_TILE, ROWS_PER_TILE)])

    @pl.when(c == 0)
    def _():
        run(0, 0)

    @pl.when(c == 1)
    def _():
        run(1, 1)


_half = jax.ShapeDtypeStruct((NP, DH), jnp.float32)


@functools.partial(
    pl.kernel,
    out_type=(_half,) * (2 * HALVES),
    mesh=_mesh,
    scratch_types=[
        pltpu.VMEM((NCHUNK, CHUNK), jnp.int32),
        pltpu.VMEM((NCHUNK, CHUNK), jnp.int32),
        pltpu.VMEM((SLOTS, CHUNK, DH), jnp.float32),
        pltpu.VMEM_SHARED((NP, DH), jnp.float32),
    ] + [pltpu.SemaphoreType.DMA] * (2 * SLOTS),
    compiler_params=_sc_params,
)
def _prop_kernel(*refs):
    xo = refs[0:HALVES]
    xi = refs[HALVES:2 * HALVES]
    ei3, zrows = refs[2 * HALVES], refs[2 * HALVES + 1]
    to = refs[2 * HALVES + 2:3 * HALVES + 2]
    ti = refs[3 * HALVES + 2:4 * HALVES + 2]
    idx_src, idx_dst, rows, accum = refs[4 * HALVES + 2:4 * HALVES + 6]
    gsem = refs[4 * HALVES + 6:4 * HALVES + 6 + SLOTS]
    ssem = refs[4 * HALVES + 6 + SLOTS:]
    c = lax.axis_index("c")
    s = lax.axis_index("s")

    def run(tables, src_row, dst_row, outs):
        pltpu.async_copy(ei3.at[src_row, s], idx_src, gsem[0])
        pltpu.async_copy(ei3.at[dst_row, s], idx_dst, gsem[1])
        for half, (table, out) in enumerate(zip(tables, outs)):
            for k in range(ROWS_PER_TILE // ZCHUNK):
                pltpu.sync_copy(
                    zrows, accum.at[pl.ds(s * ROWS_PER_TILE + k * ZCHUNK, ZCHUNK)])
            if half == 0:
                pltpu.make_async_copy(ei3.at[src_row, s], idx_src, gsem[0]).wait()
                pltpu.make_async_copy(ei3.at[dst_row, s], idx_dst, gsem[1]).wait()
            plsc.subcore_barrier()
            for k in range(SLOTS):
                pltpu.async_copy(table.at[idx_src.at[k]], rows.at[k], gsem[k])

            def group(j, carry):
                base = SLOTS * j
                for k in range(SLOTS):
                    i = base + k
                    pltpu.make_async_copy(
                        table.at[idx_src.at[i]], rows.at[k], gsem[k]).wait()
                    pltpu.async_copy(
                        rows.at[k], accum.at[idx_dst.at[i]], ssem[k], add=True)
                for k in range(SLOTS):
                    i = base + k

                    @pl.when(i + SLOTS < NCHUNK)
                    def _():
                        pltpu.make_async_copy(
                            rows.at[k], accum.at[idx_dst.at[i]], ssem[k]).wait()
                        pltpu.async_copy(
                            table.at[idx_src.at[i + SLOTS]], rows.at[k], gsem[k])

                return carry

            lax.fori_loop(0, NCHUNK // SLOTS, group, 0)
            for k in range(SLOTS):
                i = NCHUNK - SLOTS + k
                pltpu.make_async_copy(
                    rows.at[k], accum.at[idx_dst.at[i]], ssem[k]).wait()
            plsc.subcore_barrier()
            pltpu.sync_copy(accum.at[pl.ds(s * ROWS_PER_TILE, ROWS_PER_TILE)],
                            out.at[pl.ds(s * ROWS_PER_TILE, ROWS_PER_TILE)])

    @pl.when(c == 0)
    def _():
        run(xo, 0, 1, to)

    @pl.when(c == 1)
    def _():
        run(xi, 1, 0, ti)


def _inv(d):
    return jnp.where(d > 0, 1.0 / d, 0.0)


def _split_store(refs, val):
    for h, ref in enumerate(refs):
        ref[...] = val[:, h * DH:(h + 1) * DH]


def _prescale_body(x_ref, do_ref, di_ref, *out_refs):
    xv = x_ref[...]
    _split_store(out_refs[:HALVES], xv * _inv(do_ref[...]))
    _split_store(out_refs[HALVES:], xv * _inv(di_ref[...]))


def _gates(x_ref, t_refs, w_ref, b_ref):
    a = jnp.concatenate([x_ref[...]] + [t[...] for t in t_refs], axis=1)
    u = jnp.dot(a, w_ref[...], preferred_element_type=jnp.float32,
                precision=jax.lax.Precision.HIGHEST) + b_ref[...]
    z = jax.nn.sigmoid(u[:, :D])
    h = jnp.tanh(u[:, D:])
    return (1.0 - z) * h


def _gate_mid_body(*refs):
    x_ref = refs[0]
    t_refs = refs[1:2 * HALVES + 1]
    w_ref, b_ref, do_ref, di_ref = refs[2 * HALVES + 1:2 * HALVES + 5]
    xn_ref = refs[2 * HALVES + 5]
    out_refs = refs[2 * HALVES + 6:]
    xn = _gates(x_ref, t_refs, w_ref, b_ref)
    xn_ref[...] = xn
    _split_store(out_refs[:HALVES], xn * _inv(do_ref[...]))
    _split_store(out_refs[HALVES:], xn * _inv(di_ref[...]))


def _gate_final_body(*refs):
    x_ref = refs[0]
    t_refs = refs[1:2 * HALVES + 1]
    w_ref, b_ref, xn_ref = refs[2 * HALVES + 1:]
    xn_ref[...] = _gates(x_ref, t_refs, w_ref, b_ref)


_rows_spec = pl.BlockSpec((RBLK, D), lambda i: (i, 0))
_hrows_spec = pl.BlockSpec((RBLK, DH), lambda i: (i, 0))
_col_spec = pl.BlockSpec((RBLK, 1), lambda i: (i, 0))
_w_spec = pl.BlockSpec((3 * D, 2 * D), lambda i: (0, 0))
_b_spec = pl.BlockSpec((1, 2 * D), lambda i: (0, 0))
_f32 = jnp.float32
_half_out = jax.ShapeDtypeStruct((N, DH), _f32)
_full_out = jax.ShapeDtypeStruct((N, D), _f32)

_prescale_call = pl.pallas_call(
    _prescale_body,
    grid=(N // RBLK,),
    in_specs=[_rows_spec, _col_spec, _col_spec],
    out_specs=(_hrows_spec,) * (2 * HALVES),
    out_shape=(_half_out,) * (2 * HALVES),
)

_gate_mid_call = pl.pallas_call(
    _gate_mid_body,
    grid=(N // RBLK,),
    in_specs=[_rows_spec] + [_hrows_spec] * (2 * HALVES)
             + [_w_spec, _b_spec, _col_spec, _col_spec],
    out_specs=(_rows_spec,) + (_hrows_spec,) * (2 * HALVES),
    out_shape=(_full_out,) + (_half_out,) * (2 * HALVES),
)

_gate_final_call = pl.pallas_call(
    _gate_final_body,
    grid=(N // RBLK,),
    in_specs=[_rows_spec] + [_hrows_spec] * (2 * HALVES) + [_w_spec, _b_spec],
    out_specs=_rows_spec,
    out_shape=_full_out,
)


def _pack_weights(Wz, bz, Wh, bh):
    wz = jnp.concatenate([Wz[0, 0, :D] + Wz[1, 0, :D], Wz[0, 1, :D], Wz[1, 1, :D]], axis=0)
    wh = jnp.concatenate([Wh[0, 0, :D] + Wh[1, 0, :D], Wh[0, 1, :D], Wh[1, 1, :D]], axis=0)
    w = jnp.concatenate([wz, wh], axis=1)
    b = jnp.concatenate([bz, bh])[None, :]
    return w, b


def kernel(x, edge_index, W0_z, b0_z, W0_r, b0_r, W0_h, b0_h,
           W1_z, b1_z, W1_r, b1_r, W1_h, b1_h):
    ei3 = edge_index.astype(jnp.int32).reshape(2, NUM_TILES, NCHUNK, CHUNK)
    w0, bc0 = _pack_weights(W0_z, b0_z, W0_h, b0_h)
    w1, bc1 = _pack_weights(W1_z, b1_z, W1_h, b1_h)

    zeros_col = jnp.zeros((ROWS_PER_TILE,), jnp.float32)
    ones_col = jnp.ones((CHUNK,), jnp.float32)
    zrows = jnp.zeros((ZCHUNK, DH), jnp.float32)

    deg2 = _deg_kernel(ei3, zeros_col, ones_col)
    dego = deg2[0].reshape(NP, 1)
    degi = deg2[1].reshape(NP, 1)

    xs1 = _prescale_call(x, dego, degi)
    t1 = _prop_kernel(*xs1, ei3, zrows)
    x1, *xs2 = _gate_mid_call(x, *t1, w0, bc0, dego, degi)
    t2 = _prop_kernel(*xs2, ei3, zrows)
    x2 = _gate_final_call(x1, *t2, w1, bc1)
    return x2
